# Initial kernel scaffold; baseline (speedup 1.0000x reference)
import jax
import jax.numpy as jnp
from jax.experimental import pallas as pl


def kernel(game_x, state_x, edge_index_v_v, edge_index_history_v_s, edge_attr_history_v_s, edge_index_in_v_s, edge_index_s_s, conv1_W, conv1_b, conv2_W, conv2_b, conv3_Wrel, conv3_brel, conv3_Wroot, conv4_Wl, conv4_bl, conv4_Wr, lin_W, lin_b):
    raise NotImplementedError("write your pallas kernel here")


# speedup vs baseline: 2.8960x; 2.8960x over previous
import functools

import jax
import jax.numpy as jnp
from jax import lax
from jax.experimental import pallas as pl
from jax.experimental.pallas import tpu as pltpu
from jax.experimental.pallas import tpu_sc as plsc

N = 50000
E = 800000
H = 64

NC = 2
NS = 16
NW = NC * NS

NPAD = 50176
RPT = NPAD // NS

EPW = E // NW
BE = 1000
NCH_E = EPW // BE

EPS = E // NS
BF = 400
NCH_F = EPS // BF

_mesh = plsc.VectorSubcoreMesh(core_axis_name="c", subcore_axis_name="s")
_sc_params = pltpu.CompilerParams(use_tc_tiling_on_sc=False)


@functools.partial(
    pl.kernel,
    out_type=jax.ShapeDtypeStruct((NC, NPAD, 8), jnp.float32),
    mesh=_mesh,
    compiler_params=_sc_params,
    scratch_types=[
        pltpu.VMEM((BE,), jnp.int32),
        pltpu.VMEM((BE,), jnp.int32),
        pltpu.VMEM((BE, 8), jnp.float32),
        pltpu.VMEM_SHARED((NPAD, 8), jnp.float32),
        pltpu.SemaphoreType.DMA,
    ],
)
def _sc_seg8(x8, row, col, zeros8, out, ri_v, ci_v, rows_v, acc, sem):
  c = lax.axis_index("c")
  s = lax.axis_index("s")
  wid = s * NC + c
  pltpu.sync_copy(zeros8, acc.at[pl.ds(s * RPT, RPT)])
  plsc.subcore_barrier()

  def chunk(i):
    base = wid * EPW + i * BE
    pltpu.sync_copy(row.at[pl.ds(base, BE)], ri_v)
    pltpu.sync_copy(col.at[pl.ds(base, BE)], ci_v)
    pltpu.async_copy(x8.at[ri_v], rows_v, sem).wait()
    pltpu.sync_copy(rows_v, acc.at[ci_v], add=True)

  pl.loop(0, NCH_E)(chunk)
  plsc.subcore_barrier()
  sl = pl.ds(s * RPT, RPT)
  pltpu.sync_copy(acc.at[sl], out.at[c, sl])


@functools.partial(
    pl.kernel,
    out_type=jax.ShapeDtypeStruct((4, NPAD, 16), jnp.float32),
    mesh=_mesh,
    compiler_params=_sc_params,
    scratch_types=[
        pltpu.VMEM((BF,), jnp.int32),
        pltpu.VMEM((BF,), jnp.int32),
        pltpu.VMEM((BF,), jnp.float32),
        pltpu.VMEM((16,), jnp.int32),
        pltpu.VMEM((BF, 16), jnp.float32),
        pltpu.VMEM_SHARED((NPAD, 16), jnp.float32),
        pltpu.SemaphoreType.DMA,
    ],
)
def _sc_seg16(xq, row, col, w, flag, zeros16, out,
              ri_v, ci_v, w_v, fl_v, rows_v, acc, sem):
  c = lax.axis_index("c")
  s = lax.axis_index("s")
  pltpu.sync_copy(flag, fl_v)
  f = fl_v[pl.ds(0, 16)][0]
  for p in range(2):
    q = c * 2 + p
    pltpu.sync_copy(zeros16, acc.at[pl.ds(s * RPT, RPT)])
    plsc.subcore_barrier()

    def chunk(i):
      base = s * EPS + i * BF
      pltpu.sync_copy(row.at[pl.ds(base, BF)], ri_v)
      pltpu.sync_copy(col.at[pl.ds(base, BF)], ci_v)
      off = q * N
      for k in range(BF // 16):
        ksl = pl.ds(k * 16, 16)
        ri_v[ksl] = ri_v[ksl] + off
      pltpu.async_copy(xq.at[ri_v], rows_v, sem).wait()

      @pl.when(f == 1)
      def _():
        pltpu.sync_copy(w.at[pl.ds(base, BF)], w_v)

        def scale(g):
          w16 = w_v[pl.ds(g * 16, 16)]
          for j in range(16):
            e = g * 16 + j
            esl = pl.ds(0, 16)
            rows_v[e, esl] = rows_v[e, esl] * w16[j]

        pl.loop(0, BF // 16)(scale)

      pltpu.sync_copy(rows_v, acc.at[ci_v], add=True)

    pl.loop(0, NCH_F)(chunk)
    plsc.subcore_barrier()
    osl = pl.ds(s * RPT, RPT)
    pltpu.sync_copy(acc.at[osl], out.at[q, osl])
    plsc.subcore_barrier()


R = 2000
NB = N // R


def _dis(deg):
  return jnp.where(deg > 0, lax.rsqrt(jnp.maximum(deg, 1e-12)), 0.0)


def _mm(a, w):
  return jnp.dot(a, w, preferred_element_type=jnp.float32)


_row8 = pl.BlockSpec((R, 8), lambda i: (i, 0))
_row64 = pl.BlockSpec((R, H), lambda i: (i, 0))
_pair8 = pl.BlockSpec((NC, R, 8), lambda i: (0, i, 0))
_quad16 = pl.BlockSpec((4, R, 16), lambda i: (0, i, 0))


def _full(shape, ng=1):
  if ng == 1:
    return pl.BlockSpec(shape, lambda i: tuple(0 for _ in shape))
  return pl.BlockSpec(shape, lambda i, q: tuple(0 for _ in shape))


def _tc_prep_body(hv, hi, hs, gx8, disg_r, xs1_r, diss_r, invc_r):
  dg = _dis(hv[0, :, :1] + hv[1, :, :1])
  disg_r[...] = jnp.broadcast_to(dg, (R, 8))
  xs1_r[...] = dg * gx8[...]
  ds_ = _dis(hs[0, :, :1] + hs[1, :, :1])
  diss_r[...] = jnp.broadcast_to(ds_, (R, 8))
  cnt = hi[0, :, :1] + hi[1, :, :1]
  invc_r[...] = jnp.broadcast_to(1.0 / jnp.maximum(cnt, 1.0), (R, 8))


def _tc_prep(hv, hi, hs, gx8):
  return pl.pallas_call(
      _tc_prep_body,
      grid=(NB,),
      in_specs=[_pair8] * 3 + [_row8],
      out_specs=[_row8] * 4,
      out_shape=[jax.ShapeDtypeStruct((N, 8), jnp.float32)] * 4,
  )(hv, hi, hs, gx8)


def _tc_rescale8_body(qp, disg, p1_r, xs2_r):
  p1 = disg[...] * (qp[0] + qp[1])
  p1_r[...] = p1
  xs2_r[...] = disg[...] * p1


def _tc_rescale8(qp, disg):
  return pl.pallas_call(
      _tc_rescale8_body,
      grid=(NB,),
      in_specs=[_pair8, _row8],
      out_specs=[_row8] * 2,
      out_shape=[jax.ShapeDtypeStruct((N, 8), jnp.float32)] * 2,
  )(qp, disg)


def _tc_conv1_body(qp, disg, gx8, p1, w0, w1, w2, b, gq_r):
  p2 = disg[...] * (qp[0] + qp[1])
  g = _mm(gx8[...], w0[0]) + _mm(p1[...], w1[0]) + _mm(p2, w2[0])
  gq_r[0] = jnp.maximum(g + b[0], 0.0)


def _tc_conv1(qp, disg, gx8, p1, w0, w1, w2, b):
  pair = pl.BlockSpec((NC, R, 8), lambda i, q: (0, i, 0))
  row = pl.BlockSpec((R, 8), lambda i, q: (i, 0))
  wq = pl.BlockSpec((1, 8, 16), lambda i, q: (q, 0, 0))
  bq = pl.BlockSpec((1, 1, 16), lambda i, q: (q, 0, 0))
  return pl.pallas_call(
      _tc_conv1_body,
      grid=(NB, 4),
      in_specs=[pair, row, row, row, wq, wq, wq, bq],
      out_specs=pl.BlockSpec((1, R, 16), lambda i, q: (q, i, 0)),
      out_shape=jax.ShapeDtypeStruct((4, N, 16), jnp.float32),
  )(qp, disg, gx8, p1, w0, w1, w2, b)


def _cat4(x):
  return jnp.concatenate([x[0], x[1], x[2], x[3]], axis=1)


def _stack4(x):
  return jnp.stack([x[:, k * 16:(k + 1) * 16] for k in range(4)])


def _tc_conv34_body(ag, ss, invc, sx8, wrel, brel, wroot, wl, bl, wr, s4_r):
  agg = _cat4(ag[...])
  s3 = jnp.maximum(_mm(agg, wrel[...]) + brel[...]
                   + _mm(sx8[...], wroot[...]), 0.0)
  mean = _cat4(ss[...]) * invc[:, :1]
  s4 = jnp.maximum(_mm(mean, wl[...]) + bl[...] + _mm(s3, wr[...]), 0.0)
  s4_r[...] = _stack4(s4)


def _tc_conv34(ag, ss, invc, sx8, wrel, brel, wroot, wl, bl, wr):
  return pl.pallas_call(
      _tc_conv34_body,
      grid=(NB,),
      in_specs=[_quad16, _quad16, _row8, _row8,
                _full((H, H)), _full((1, H)), _full((8, H)),
                _full((H, H)), _full((1, H)), _full((H, H))],
      out_specs=pl.BlockSpec((4, R, 16), lambda i: (0, i, 0)),
      out_shape=jax.ShapeDtypeStruct((4, N, 16), jnp.float32),
  )(ag, ss, invc, sx8, wrel, brel, wroot, wl, bl, wr)


def _tc_t_dense_body(sq, diss, t_r):
  t_r[0] = diss[:, :1] * sq[0]


def _tc_t_dense(sq, diss):
  return pl.pallas_call(
      _tc_t_dense_body,
      grid=(NB, 4),
      in_specs=[pl.BlockSpec((1, R, 16), lambda i, q: (q, i, 0)),
                pl.BlockSpec((R, 8), lambda i, q: (i, 0))],
      out_specs=pl.BlockSpec((1, R, 16), lambda i, q: (q, i, 0)),
      out_shape=jax.ShapeDtypeStruct((4, N, 16), jnp.float32),
  )(sq, diss)


def _tc_t_quart_body(rq, diss, t_r):
  d = diss[:, :1]
  t_r[0] = d * d * rq[0]


def _tc_t_quart(rq, diss):
  return pl.pallas_call(
      _tc_t_quart_body,
      grid=(NB, 4),
      in_specs=[pl.BlockSpec((1, R, 16), lambda i, q: (q, i, 0)),
                pl.BlockSpec((R, 8), lambda i, q: (i, 0))],
      out_specs=pl.BlockSpec((1, R, 16), lambda i, q: (q, i, 0)),
      out_shape=jax.ShapeDtypeStruct((4, N, 16), jnp.float32),
  )(rq, diss)


def _tc_tag0_body(rq, diss, s4q, w20, w21, acc_r):
  u = diss[:, :1] * _cat4(rq[...])
  acc_r[...] = _mm(_cat4(s4q[...]), w20[...]) + _mm(u, w21[...])


def _tc_tag0(rq, diss, s4q, w20, w21):
  s4spec = pl.BlockSpec((4, R, 16), lambda i: (0, i, 0))
  return pl.pallas_call(
      _tc_tag0_body,
      grid=(NB,),
      in_specs=[_quad16, _row8, s4spec, _full((H, H)), _full((H, H))],
      out_specs=_row64,
      out_shape=jax.ShapeDtypeStruct((N, H), jnp.float32),
  )(rq, diss, s4q, w20, w21)


def _tc_tag1_body(rq, diss, acc, w2k, acc_r):
  u = diss[:, :1] * _cat4(rq[...])
  acc_r[...] = acc[...] + _mm(u, w2k[...])


def _tc_tag1(rq, diss, acc, w2k):
  return pl.pallas_call(
      _tc_tag1_body,
      grid=(NB,),
      in_specs=[_quad16, _row8, _row64, _full((H, H))],
      out_specs=_row64,
      out_shape=jax.ShapeDtypeStruct((N, H), jnp.float32),
  )(rq, diss, acc, w2k)


def _tc_final_body(rq, diss, acc, w23, b2, linw, linb, out_r):
  u = diss[:, :1] * _cat4(rq[...])
  s = jnp.maximum(acc[...] + _mm(u, w23[...]) + b2[...], 0.0)
  out_r[...] = _mm(s, linw[...]) + linb[...]


def _tc_final(rq, diss, acc, w23, b2, linw, linb):
  return pl.pallas_call(
      _tc_final_body,
      grid=(NB,),
      in_specs=[_quad16, _row8, _row64, _full((H, H)), _full((1, H)),
                _full((H, 8)), _full((1, 8))],
      out_specs=_row8,
      out_shape=jax.ShapeDtypeStruct((N, 8), jnp.float32),
  )(rq, diss, acc, w23, b2, linw, linb)


def kernel(game_x, state_x, edge_index_v_v, edge_index_history_v_s,
           edge_attr_history_v_s, edge_index_in_v_s, edge_index_s_s,
           conv1_W, conv1_b, conv2_W, conv2_b,
           conv3_Wrel, conv3_brel, conv3_Wroot,
           conv4_Wl, conv4_bl, conv4_Wr, lin_W, lin_b):
  f32 = jnp.float32
  i32 = jnp.int32
  gx8 = jnp.pad(game_x.astype(f32), ((0, 0), (0, 3)))
  sx8 = jnp.pad(state_x.astype(f32), ((0, 0), (0, 2)))

  row_vv = edge_index_v_v[0].astype(i32)
  col_vv = edge_index_v_v[1].astype(i32)
  row_h = edge_index_history_v_s[0].astype(i32)
  col_h = edge_index_history_v_s[1].astype(i32)
  row_in = edge_index_in_v_s[0].astype(i32)
  col_in = edge_index_in_v_s[1].astype(i32)
  row_ss = edge_index_s_s[0].astype(i32)
  col_ss = edge_index_s_s[1].astype(i32)

  w1p = jnp.pad(conv1_W.astype(f32), ((0, 0), (0, 3), (0, 0)))
  w1q = [w1p[k].reshape(8, 4, 16).transpose(1, 0, 2) for k in range(3)]
  b1q = conv1_b.astype(f32).reshape(4, 1, 16)
  wrootp = jnp.pad(conv3_Wroot.astype(f32), ((0, 2), (0, 0)))
  w2 = conv2_W.astype(f32)

  z8 = jnp.zeros((RPT, 8), f32)
  z16 = jnp.zeros((RPT, 16), f32)
  ones_tab = jnp.ones((N, 8), f32)
  zrow = jnp.zeros((E,), i32)
  wzero = jnp.zeros((E,), f32)
  flag0 = jnp.zeros((16,), i32)
  flag1 = jnp.ones((16,), i32)

  hv = _sc_seg8(ones_tab, zrow, col_vv, z8)
  hi = _sc_seg8(ones_tab, zrow, col_in, z8)
  hs = _sc_seg8(ones_tab, zrow, col_ss, z8)
  disg, xs1, diss, invc = _tc_prep(hv, hi, hs, gx8)

  q1 = _sc_seg8(xs1, row_vv, col_vv, z8)
  p1, xs2 = _tc_rescale8(q1, disg)
  q2 = _sc_seg8(xs2, row_vv, col_vv, z8)
  gq3 = _tc_conv1(q2, disg, gx8, p1, w1q[0], w1q[1], w1q[2], b1q)
  gq = gq3.reshape(4 * N, 16)

  ag = _sc_seg16(gq, row_h, col_h, edge_attr_history_v_s.astype(f32),
                 flag1, z16)
  ss = _sc_seg16(gq, row_in, col_in, wzero, flag0, z16)
  s4q = _tc_conv34(ag, ss, invc, sx8,
                   conv3_Wrel.astype(f32),
                   conv3_brel.astype(f32).reshape(1, H), wrootp,
                   conv4_Wl.astype(f32), conv4_bl.astype(f32).reshape(1, H),
                   conv4_Wr.astype(f32))

  t1q = _tc_t_dense(s4q, diss).reshape(4 * N, 16)
  r1 = _sc_seg16(t1q, row_ss, col_ss, wzero, flag0, z16)
  acc = _tc_tag0(r1, diss, s4q, w2[0], w2[1])
  t2q = _tc_t_quart(r1, diss).reshape(4 * N, 16)
  r2 = _sc_seg16(t2q, row_ss, col_ss, wzero, flag0, z16)
  acc = _tc_tag1(r2, diss, acc, w2[2])
  t3q = _tc_t_quart(r2, diss).reshape(4 * N, 16)
  r3 = _sc_seg16(t3q, row_ss, col_ss, wzero, flag0, z16)
  return _tc_final(r3, diss, acc, w2[3],
                   conv2_b.astype(f32).reshape(1, H),
                   lin_W.astype(f32), lin_b.astype(f32).reshape(1, 8))


# speedup vs baseline: 13.4423x; 4.6417x over previous
import functools

import jax
import jax.numpy as jnp
from jax import lax
from jax.experimental import pallas as pl
from jax.experimental.pallas import tpu as pltpu
from jax.experimental.pallas import tpu_sc as plsc

N = 50000
E = 800000
H = 64

NC = 2
NS = 16
NW = NC * NS

NPAD = 50176
RPT = NPAD // NS

EPAD = 802816
EP = EPAD - E
EHALF = EPAD // 2
SPW = EPAD // NW
SPS = EPAD // NS
BF = 512
NCHB = SPW // BF
NCHA = SPS // BF

_mesh = plsc.VectorSubcoreMesh(core_axis_name="c", subcore_axis_name="s")
_sc_params = pltpu.CompilerParams(use_tc_tiling_on_sc=False)


@functools.partial(
    pl.kernel,
    out_type=jax.ShapeDtypeStruct((NC, NPAD, 32), jnp.float32),
    mesh=_mesh,
    compiler_params=_sc_params,
    scratch_types=[
        pltpu.VMEM((BF,), jnp.int32),
        pltpu.VMEM((BF,), jnp.int32),
        pltpu.VMEM((BF,), jnp.float32),
        pltpu.VMEM((16,), jnp.int32),
        pltpu.VMEM((BF, 32), jnp.float32),
        pltpu.VMEM_SHARED((NPAD, 32), jnp.float32),
        pltpu.SemaphoreType.DMA,
    ],
)
def _sc_seg(x, rowcat, col, w, flag, zeros, out,
            ri_v, ci_v, w_v, fl_v, rows_v, acc, sem):
  c = lax.axis_index("c")
  s = lax.axis_index("s")
  pltpu.sync_copy(flag, fl_v)
  fv = fl_v[pl.ds(0, 16)]
  wf = fv[0]
  ma = fv[1]
  hf = fv[2]
  ebc = (1 - ma) * EHALF
  sb = ma * SPS + (1 - ma) * SPW
  rb = ma * EPAD
  pltpu.sync_copy(zeros, acc.at[pl.ds(s * RPT, RPT)])

  @pl.when(hf == 1)
  def _():
    pltpu.sync_copy(x.at[pl.ds(0, BF)], rows_v)

  plsc.subcore_barrier()

  def chunk(i):
    base = c * ebc + s * sb + i * BF

    @pl.when(hf == 0)
    def _():
      pltpu.sync_copy(rowcat.at[pl.ds(c * rb + base, BF)], ri_v)
      pltpu.async_copy(x.at[ri_v], rows_v, sem).wait()

    pltpu.sync_copy(col.at[pl.ds(base, BF)], ci_v)

    @pl.when(wf == 1)
    def _():
      pltpu.sync_copy(w.at[pl.ds(base, BF)], w_v)

      def scale(g):
        w16 = w_v[pl.ds(g * 16, 16)]
        for j in range(16):
          e = g * 16 + j
          lo = pl.ds(0, 16)
          hi = pl.ds(16, 16)
          rows_v[e, lo] = rows_v[e, lo] * w16[j]
          rows_v[e, hi] = rows_v[e, hi] * w16[j]

      pl.loop(0, BF // 16)(scale)

    pltpu.sync_copy(rows_v, acc.at[ci_v], add=True)

  pl.loop(0, NCHB)(chunk)

  @pl.when(ma == 1)
  def _():
    pl.loop(NCHB, NCHA)(chunk)

  plsc.subcore_barrier()
  sl = pl.ds(s * RPT, RPT)
  pltpu.sync_copy(acc.at[sl], out.at[c, sl])


R = 2000
NB = N // R


def _dis(deg):
  return jnp.where(deg > 0, lax.rsqrt(jnp.maximum(deg, 1e-12)), 0.0)


def _mm(a, w):
  return jnp.dot(a, w, preferred_element_type=jnp.float32)


_row8 = pl.BlockSpec((R, 8), lambda i: (i, 0))
_row64 = pl.BlockSpec((R, H), lambda i: (i, 0))
_half32 = pl.BlockSpec((NC, R, 32), lambda i: (0, i, 0))
_src32 = pl.BlockSpec((R, 32), lambda i: (i, 0))


def _full(shape, ng=1):
  if ng == 1:
    return pl.BlockSpec(shape, lambda i: tuple(0 for _ in shape))
  return pl.BlockSpec(shape, lambda i, q: tuple(0 for _ in shape))


def _cat2(x):
  return jnp.concatenate([x[0], x[1]], axis=1)


def _pad32(x8):
  return jnp.concatenate([x8, jnp.zeros((R, 24), jnp.float32)], axis=1)


def _tc_prep_body(hv, hi, hs, gx8, disg_r, xs1_r, diss_r, invc_r):
  dg = _dis(hv[0, :, :1] + hv[1, :, :1])
  disg_r[...] = jnp.broadcast_to(dg, (R, 8))
  xs1_r[...] = _pad32(dg * gx8[...])
  ds_ = _dis(hs[0, :, :1] + hs[1, :, :1])
  diss_r[...] = jnp.broadcast_to(ds_, (R, 8))
  cnt = hi[0, :, :1] + hi[1, :, :1]
  invc_r[...] = jnp.broadcast_to(1.0 / jnp.maximum(cnt, 1.0), (R, 8))


def _tc_prep(hv, hi, hs, gx8):
  return pl.pallas_call(
      _tc_prep_body,
      grid=(NB,),
      in_specs=[_half32] * 3 + [_row8],
      out_specs=[_row8, _src32, _row8, _row8],
      out_shape=[jax.ShapeDtypeStruct((N, 8), jnp.float32),
                 jax.ShapeDtypeStruct((2 * N, 32), jnp.float32),
                 jax.ShapeDtypeStruct((N, 8), jnp.float32),
                 jax.ShapeDtypeStruct((N, 8), jnp.float32)],
  )(hv, hi, hs, gx8)


def _tc_rescale8_body(qp, disg, p1_r, xs2_r):
  p1 = disg[...] * (qp[0, :, :8] + qp[1, :, :8])
  p1_r[...] = p1
  xs2_r[...] = _pad32(disg[...] * p1)


def _tc_rescale8(qp, disg):
  return pl.pallas_call(
      _tc_rescale8_body,
      grid=(NB,),
      in_specs=[_half32, _row8],
      out_specs=[_row8, _src32],
      out_shape=[jax.ShapeDtypeStruct((N, 8), jnp.float32),
                 jax.ShapeDtypeStruct((2 * N, 32), jnp.float32)],
  )(qp, disg)


def _tc_conv1_body(qp, disg, gx8, p1, w0, w1, w2, b, gh_r):
  p2 = disg[...] * (qp[0, :, :8] + qp[1, :, :8])
  g = _mm(gx8[...], w0[0]) + _mm(p1[...], w1[0]) + _mm(p2, w2[0])
  gh_r[0] = jnp.maximum(g + b[0], 0.0)


def _tc_conv1(qp, disg, gx8, p1, w0, w1, w2, b):
  pair = pl.BlockSpec((NC, R, 32), lambda i, q: (0, i, 0))
  row = pl.BlockSpec((R, 8), lambda i, q: (i, 0))
  wh = pl.BlockSpec((1, 8, 32), lambda i, q: (q, 0, 0))
  bh = pl.BlockSpec((1, 1, 32), lambda i, q: (q, 0, 0))
  return pl.pallas_call(
      _tc_conv1_body,
      grid=(NB, 2),
      in_specs=[pair, row, row, row, wh, wh, wh, bh],
      out_specs=pl.BlockSpec((1, R, 32), lambda i, q: (q, i, 0)),
      out_shape=jax.ShapeDtypeStruct((2, N, 32), jnp.float32),
  )(qp, disg, gx8, p1, w0, w1, w2, b)


def _tc_conv34_body(ag, ss, invc, sx8, wrel, brel, wroot, wl, bl, wr, s4_r):
  agg = _cat2(ag[...])
  s3 = jnp.maximum(_mm(agg, wrel[...]) + brel[...]
                   + _mm(sx8[...], wroot[...]), 0.0)
  mean = _cat2(ss[...]) * invc[:, :1]
  s4 = jnp.maximum(_mm(mean, wl[...]) + bl[...] + _mm(s3, wr[...]), 0.0)
  s4_r[...] = jnp.stack([s4[:, :32], s4[:, 32:]])


def _tc_conv34(ag, ss, invc, sx8, wrel, brel, wroot, wl, bl, wr):
  return pl.pallas_call(
      _tc_conv34_body,
      grid=(NB,),
      in_specs=[_half32, _half32, _row8, _row8,
                _full((H, H)), _full((1, H)), _full((8, H)),
                _full((H, H)), _full((1, H)), _full((H, H))],
      out_specs=pl.BlockSpec((2, R, 32), lambda i: (0, i, 0)),
      out_shape=jax.ShapeDtypeStruct((2, N, 32), jnp.float32),
  )(ag, ss, invc, sx8, wrel, brel, wroot, wl, bl, wr)


def _tc_t_dense_body(sh, diss, t_r):
  t_r[0] = diss[:, :1] * sh[0]


def _tc_t_dense(sh, diss):
  return pl.pallas_call(
      _tc_t_dense_body,
      grid=(NB, 2),
      in_specs=[pl.BlockSpec((1, R, 32), lambda i, q: (q, i, 0)),
                pl.BlockSpec((R, 8), lambda i, q: (i, 0))],
      out_specs=pl.BlockSpec((1, R, 32), lambda i, q: (q, i, 0)),
      out_shape=jax.ShapeDtypeStruct((2, N, 32), jnp.float32),
  )(sh, diss)


def _tc_t_half_body(rh, diss, t_r):
  d = diss[:, :1]
  t_r[0] = d * d * rh[0]


def _tc_t_half(rh, diss):
  return pl.pallas_call(
      _tc_t_half_body,
      grid=(NB, 2),
      in_specs=[pl.BlockSpec((1, R, 32), lambda i, q: (q, i, 0)),
                pl.BlockSpec((R, 8), lambda i, q: (i, 0))],
      out_specs=pl.BlockSpec((1, R, 32), lambda i, q: (q, i, 0)),
      out_shape=jax.ShapeDtypeStruct((2, N, 32), jnp.float32),
  )(rh, diss)


def _tc_tag0_body(rh, diss, s4h, w20, w21, acc_r):
  u = diss[:, :1] * _cat2(rh[...])
  acc_r[...] = _mm(_cat2(s4h[...]), w20[...]) + _mm(u, w21[...])


def _tc_tag0(rh, diss, s4h, w20, w21):
  s4spec = pl.BlockSpec((2, R, 32), lambda i: (0, i, 0))
  return pl.pallas_call(
      _tc_tag0_body,
      grid=(NB,),
      in_specs=[_half32, _row8, s4spec, _full((H, H)), _full((H, H))],
      out_specs=_row64,
      out_shape=jax.ShapeDtypeStruct((N, H), jnp.float32),
  )(rh, diss, s4h, w20, w21)


def _tc_tag1_body(rh, diss, acc, w2k, acc_r):
  u = diss[:, :1] * _cat2(rh[...])
  acc_r[...] = acc[...] + _mm(u, w2k[...])


def _tc_tag1(rh, diss, acc, w2k):
  return pl.pallas_call(
      _tc_tag1_body,
      grid=(NB,),
      in_specs=[_half32, _row8, _row64, _full((H, H))],
      out_specs=_row64,
      out_shape=jax.ShapeDtypeStruct((N, H), jnp.float32),
  )(rh, diss, acc, w2k)


def _tc_final_body(rh, diss, acc, w23, b2, linw, linb, out_r):
  u = diss[:, :1] * _cat2(rh[...])
  s = jnp.maximum(acc[...] + _mm(u, w23[...]) + b2[...], 0.0)
  out_r[...] = _mm(s, linw[...]) + linb[...]


def _tc_final(rh, diss, acc, w23, b2, linw, linb):
  return pl.pallas_call(
      _tc_final_body,
      grid=(NB,),
      in_specs=[_half32, _row8, _row64, _full((H, H)), _full((1, H)),
                _full((H, 8)), _full((1, 8))],
      out_specs=_row8,
      out_shape=jax.ShapeDtypeStruct((N, 8), jnp.float32),
  )(rh, diss, acc, w23, b2, linw, linb)


def kernel(game_x, state_x, edge_index_v_v, edge_index_history_v_s,
           edge_attr_history_v_s, edge_index_in_v_s, edge_index_s_s,
           conv1_W, conv1_b, conv2_W, conv2_b,
           conv3_Wrel, conv3_brel, conv3_Wroot,
           conv4_Wl, conv4_bl, conv4_Wr, lin_W, lin_b):
  f32 = jnp.float32
  i32 = jnp.int32
  gx8 = jnp.pad(game_x.astype(f32), ((0, 0), (0, 3)))
  sx8 = jnp.pad(state_x.astype(f32), ((0, 0), (0, 2)))

  padr = jnp.zeros((EP,), i32)
  padc = jnp.full((EP,), NPAD - 1, i32)

  def prow(r):
    return jnp.concatenate([r.astype(i32), padr])

  def pcol(cl):
    return jnp.concatenate([cl.astype(i32), padc])

  rvv = prow(edge_index_v_v[0])
  cvv = pcol(edge_index_v_v[1])
  rh_ = prow(edge_index_history_v_s[0])
  ch = pcol(edge_index_history_v_s[1])
  rin = prow(edge_index_in_v_s[0])
  cin = pcol(edge_index_in_v_s[1])
  rss = prow(edge_index_s_s[0])
  css = pcol(edge_index_s_s[1])

  rvvB = jnp.concatenate([rvv, rvv])
  rhA = jnp.concatenate([rh_, rh_ + N])
  rinA = jnp.concatenate([rin, rin + N])
  rssA = jnp.concatenate([rss, rss + N])
  zrow2 = jnp.zeros((2 * EPAD,), i32)

  w1p = jnp.pad(conv1_W.astype(f32), ((0, 0), (0, 3), (0, 0)))
  w1h = [w1p[k].reshape(8, 2, 32).transpose(1, 0, 2) for k in range(3)]
  b1h = conv1_b.astype(f32).reshape(2, 1, 32)
  wrootp = jnp.pad(conv3_Wroot.astype(f32), ((0, 2), (0, 0)))
  w2 = conv2_W.astype(f32)

  z32 = jnp.zeros((RPT, 32), f32)
  ones2 = jnp.ones((2 * N, 32), f32)
  wz = jnp.zeros((EPAD,), f32)
  wh_ = jnp.concatenate([edge_attr_history_v_s.astype(f32),
                         jnp.zeros((EP,), f32)])

  def mkflag(wf, ma, hf):
    return jnp.array([wf, ma, hf] + [0] * 13, i32)

  f_hist = mkflag(0, 0, 1)
  f_b = mkflag(0, 0, 0)
  f_a = mkflag(0, 1, 0)
  f_aw = mkflag(1, 1, 0)

  hv = _sc_seg(ones2, zrow2, cvv, wz, f_hist, z32)
  hi = _sc_seg(ones2, zrow2, cin, wz, f_hist, z32)
  hs = _sc_seg(ones2, zrow2, css, wz, f_hist, z32)
  disg, xs1cat, diss, invc = _tc_prep(hv, hi, hs, gx8)

  q1 = _sc_seg(xs1cat, rvvB, cvv, wz, f_b, z32)
  p1, xs2cat = _tc_rescale8(q1, disg)
  q2 = _sc_seg(xs2cat, rvvB, cvv, wz, f_b, z32)
  gq = _tc_conv1(q2, disg, gx8, p1, w1h[0], w1h[1], w1h[2], b1h)
  gqcat = gq.reshape(2 * N, 32)

  ag = _sc_seg(gqcat, rhA, ch, wh_, f_aw, z32)
  ss = _sc_seg(gqcat, rinA, cin, wz, f_a, z32)
  s4h = _tc_conv34(ag, ss, invc, sx8,
                   conv3_Wrel.astype(f32),
                   conv3_brel.astype(f32).reshape(1, H), wrootp,
                   conv4_Wl.astype(f32), conv4_bl.astype(f32).reshape(1, H),
                   conv4_Wr.astype(f32))

  t1cat = _tc_t_dense(s4h, diss).reshape(2 * N, 32)
  r1 = _sc_seg(t1cat, rssA, css, wz, f_a, z32)
  acc = _tc_tag0(r1, diss, s4h, w2[0], w2[1])
  t2cat = _tc_t_half(r1, diss).reshape(2 * N, 32)
  r2 = _sc_seg(t2cat, rssA, css, wz, f_a, z32)
  acc = _tc_tag1(r2, diss, acc, w2[2])
  t3cat = _tc_t_half(r2, diss).reshape(2 * N, 32)
  r3 = _sc_seg(t3cat, rssA, css, wz, f_a, z32)
  return _tc_final(r3, diss, acc, w2[3],
                   conv2_b.astype(f32).reshape(1, H),
                   lin_W.astype(f32), lin_b.astype(f32).reshape(1, 8))


# speedup vs baseline: 14.1580x; 1.0532x over previous
import functools

import jax
import jax.numpy as jnp
from jax import lax
from jax.experimental import pallas as pl
from jax.experimental.pallas import tpu as pltpu
from jax.experimental.pallas import tpu_sc as plsc

N = 50000
E = 800000
H = 64

NC = 2
NS = 16
NW = NC * NS

NPAD = 50176
RPT = NPAD // NS

EPAD = 802816
EP = EPAD - E
EHALF = EPAD // 2
SPW = EPAD // NW
SPS = EPAD // NS
BF = 256
NCHB = SPW // BF
NCHA = SPS // BF

_mesh = plsc.VectorSubcoreMesh(core_axis_name="c", subcore_axis_name="s")
_sc_params = pltpu.CompilerParams(use_tc_tiling_on_sc=False)


@functools.partial(
    pl.kernel,
    out_type=jax.ShapeDtypeStruct((NC, NPAD, 32), jnp.float32),
    mesh=_mesh,
    compiler_params=_sc_params,
    scratch_types=[
        pltpu.VMEM((2, BF), jnp.int32),
        pltpu.VMEM((2, BF), jnp.int32),
        pltpu.VMEM((2, BF), jnp.float32),
        pltpu.VMEM((16,), jnp.int32),
        pltpu.VMEM((2, BF, 32), jnp.float32),
        pltpu.VMEM_SHARED((NPAD, 32), jnp.float32),
        pltpu.SemaphoreType.DMA((2,)),
    ],
)
def _sc_seg(x, rowcat, col, w, flag, zeros, out,
            ri2, ci2, w2, fl_v, rows2, acc, sem2):
  c = lax.axis_index("c")
  s = lax.axis_index("s")
  pltpu.sync_copy(flag, fl_v)
  fv = fl_v[pl.ds(0, 16)]
  wf = fv[0]
  ma = fv[1]
  hf = fv[2]
  ebc = (1 - ma) * EHALF
  sb = ma * SPS + (1 - ma) * SPW
  rb = ma * EPAD
  pltpu.sync_copy(zeros, acc.at[pl.ds(s * RPT, RPT)])

  @pl.when(hf == 1)
  def _():
    pltpu.sync_copy(x.at[pl.ds(0, BF)], rows2.at[0])
    pltpu.sync_copy(x.at[pl.ds(0, BF)], rows2.at[1])

  plsc.subcore_barrier()

  def start(t, jj):
    base = c * ebc + s * sb + t * BF
    pltpu.sync_copy(col.at[pl.ds(base, BF)], ci2.at[jj])

    @pl.when(wf == 1)
    def _():
      pltpu.sync_copy(w.at[pl.ds(base, BF)], w2.at[jj])

    @pl.when(hf == 0)
    def _():
      pltpu.sync_copy(rowcat.at[pl.ds(c * rb + base, BF)], ri2.at[jj])
      pltpu.async_copy(x.at[ri2.at[jj]], rows2.at[jj], sem2.at[jj])

  def finish(jj):
    rv = rows2.at[jj]

    @pl.when(hf == 0)
    def _():
      pltpu.make_async_copy(x.at[ri2.at[jj]], rv, sem2.at[jj]).wait()

    @pl.when(wf == 1)
    def _():
      wv = w2.at[jj]

      def scale(g):
        w16 = wv[pl.ds(g * 16, 16)]
        for j in range(16):
          e = g * 16 + j
          lo = pl.ds(0, 16)
          hi = pl.ds(16, 16)
          rv[e, lo] = rv[e, lo] * w16[j]
          rv[e, hi] = rv[e, hi] * w16[j]

      pl.loop(0, BF // 16)(scale)

    pltpu.sync_copy(rv, acc.at[ci2.at[jj]], add=True)

  def body(t):
    jj = t % 2
    start(t, jj)
    finish(1 - jj)

  start(0, 0)
  pl.loop(1, NCHB)(body)

  @pl.when(ma == 1)
  def _():
    pl.loop(NCHB, NCHA)(body)

  finish(1)

  plsc.subcore_barrier()
  sl = pl.ds(s * RPT, RPT)
  pltpu.sync_copy(acc.at[sl], out.at[c, sl])


R = 2000
NB = N // R


def _dis(deg):
  return jnp.where(deg > 0, lax.rsqrt(jnp.maximum(deg, 1e-12)), 0.0)


def _mm(a, w):
  return jnp.dot(a, w, preferred_element_type=jnp.float32)


_row8 = pl.BlockSpec((R, 8), lambda i: (i, 0))
_row64 = pl.BlockSpec((R, H), lambda i: (i, 0))
_half32 = pl.BlockSpec((NC, R, 32), lambda i: (0, i, 0))
_src32 = pl.BlockSpec((R, 32), lambda i: (i, 0))


def _full(shape, ng=1):
  if ng == 1:
    return pl.BlockSpec(shape, lambda i: tuple(0 for _ in shape))
  return pl.BlockSpec(shape, lambda i, q: tuple(0 for _ in shape))


def _cat2(x):
  return jnp.concatenate([x[0], x[1]], axis=1)


def _pad32(x8):
  return jnp.concatenate([x8, jnp.zeros((R, 24), jnp.float32)], axis=1)


def _tc_prep_g_body(hv, gx8, disg_r, xs1_r):
  dg = _dis(hv[0, :, :1] + hv[1, :, :1])
  disg_r[...] = jnp.broadcast_to(dg, (R, 8))
  xs1_r[...] = _pad32(dg * gx8[...])


def _tc_prep_g(hv, gx8):
  return pl.pallas_call(
      _tc_prep_g_body,
      grid=(NB,),
      in_specs=[_half32, _row8],
      out_specs=[_row8, _src32],
      out_shape=[jax.ShapeDtypeStruct((N, 8), jnp.float32),
                 jax.ShapeDtypeStruct((2 * N, 32), jnp.float32)],
  )(hv, gx8)


def _tc_prep_si_body(hi, hs, diss_r, invc_r):
  ds_ = _dis(hs[0, :, :1] + hs[1, :, :1])
  diss_r[...] = jnp.broadcast_to(ds_, (R, 8))
  cnt = hi[0, :, :1] + hi[1, :, :1]
  invc_r[...] = jnp.broadcast_to(1.0 / jnp.maximum(cnt, 1.0), (R, 8))


def _tc_prep_si(hi, hs):
  return pl.pallas_call(
      _tc_prep_si_body,
      grid=(NB,),
      in_specs=[_half32] * 2,
      out_specs=[_row8, _row8],
      out_shape=[jax.ShapeDtypeStruct((N, 8), jnp.float32),
                 jax.ShapeDtypeStruct((N, 8), jnp.float32)],
  )(hi, hs)


def _tc_rescale8_body(qp, disg, p1_r, xs2_r):
  p1 = disg[...] * (qp[0, :, :8] + qp[1, :, :8])
  p1_r[...] = p1
  xs2_r[...] = _pad32(disg[...] * p1)


def _tc_rescale8(qp, disg):
  return pl.pallas_call(
      _tc_rescale8_body,
      grid=(NB,),
      in_specs=[_half32, _row8],
      out_specs=[_row8, _src32],
      out_shape=[jax.ShapeDtypeStruct((N, 8), jnp.float32),
                 jax.ShapeDtypeStruct((2 * N, 32), jnp.float32)],
  )(qp, disg)


def _tc_conv1_body(qp, disg, gx8, p1, w0, w1, w2, b, gh_r):
  p2 = disg[...] * (qp[0, :, :8] + qp[1, :, :8])
  g = _mm(gx8[...], w0[0]) + _mm(p1[...], w1[0]) + _mm(p2, w2[0])
  gh_r[0] = jnp.maximum(g + b[0], 0.0)


def _tc_conv1(qp, disg, gx8, p1, w0, w1, w2, b):
  pair = pl.BlockSpec((NC, R, 32), lambda i, q: (0, i, 0))
  row = pl.BlockSpec((R, 8), lambda i, q: (i, 0))
  wh = pl.BlockSpec((1, 8, 32), lambda i, q: (q, 0, 0))
  bh = pl.BlockSpec((1, 1, 32), lambda i, q: (q, 0, 0))
  return pl.pallas_call(
      _tc_conv1_body,
      grid=(NB, 2),
      in_specs=[pair, row, row, row, wh, wh, wh, bh],
      out_specs=pl.BlockSpec((1, R, 32), lambda i, q: (q, i, 0)),
      out_shape=jax.ShapeDtypeStruct((2, N, 32), jnp.float32),
  )(qp, disg, gx8, p1, w0, w1, w2, b)


def _tc_conv34_body(ag, ss, invc, sx8, wrel, brel, wroot, wl, bl, wr, s4_r):
  agg = _cat2(ag[...])
  s3 = jnp.maximum(_mm(agg, wrel[...]) + brel[...]
                   + _mm(sx8[...], wroot[...]), 0.0)
  mean = _cat2(ss[...]) * invc[:, :1]
  s4 = jnp.maximum(_mm(mean, wl[...]) + bl[...] + _mm(s3, wr[...]), 0.0)
  s4_r[...] = jnp.stack([s4[:, :32], s4[:, 32:]])


def _tc_conv34(ag, ss, invc, sx8, wrel, brel, wroot, wl, bl, wr):
  return pl.pallas_call(
      _tc_conv34_body,
      grid=(NB,),
      in_specs=[_half32, _half32, _row8, _row8,
                _full((H, H)), _full((1, H)), _full((8, H)),
                _full((H, H)), _full((1, H)), _full((H, H))],
      out_specs=pl.BlockSpec((2, R, 32), lambda i: (0, i, 0)),
      out_shape=jax.ShapeDtypeStruct((2, N, 32), jnp.float32),
  )(ag, ss, invc, sx8, wrel, brel, wroot, wl, bl, wr)


def _tc_t_dense_body(sh, diss, t_r):
  t_r[0] = diss[:, :1] * sh[0]


def _tc_t_dense(sh, diss):
  return pl.pallas_call(
      _tc_t_dense_body,
      grid=(NB, 2),
      in_specs=[pl.BlockSpec((1, R, 32), lambda i, q: (q, i, 0)),
                pl.BlockSpec((R, 8), lambda i, q: (i, 0))],
      out_specs=pl.BlockSpec((1, R, 32), lambda i, q: (q, i, 0)),
      out_shape=jax.ShapeDtypeStruct((2, N, 32), jnp.float32),
  )(sh, diss)


def _tc_t_half_body(rh, diss, t_r):
  d = diss[:, :1]
  t_r[0] = d * d * rh[0]


def _tc_t_half(rh, diss):
  return pl.pallas_call(
      _tc_t_half_body,
      grid=(NB, 2),
      in_specs=[pl.BlockSpec((1, R, 32), lambda i, q: (q, i, 0)),
                pl.BlockSpec((R, 8), lambda i, q: (i, 0))],
      out_specs=pl.BlockSpec((1, R, 32), lambda i, q: (q, i, 0)),
      out_shape=jax.ShapeDtypeStruct((2, N, 32), jnp.float32),
  )(rh, diss)


def _tc_tag0_body(rh, diss, s4h, w20, w21, acc_r):
  u = diss[:, :1] * _cat2(rh[...])
  acc_r[...] = _mm(_cat2(s4h[...]), w20[...]) + _mm(u, w21[...])


def _tc_tag0(rh, diss, s4h, w20, w21):
  s4spec = pl.BlockSpec((2, R, 32), lambda i: (0, i, 0))
  return pl.pallas_call(
      _tc_tag0_body,
      grid=(NB,),
      in_specs=[_half32, _row8, s4spec, _full((H, H)), _full((H, H))],
      out_specs=_row64,
      out_shape=jax.ShapeDtypeStruct((N, H), jnp.float32),
  )(rh, diss, s4h, w20, w21)


def _tc_tag1_body(rh, diss, acc, w2k, acc_r):
  u = diss[:, :1] * _cat2(rh[...])
  acc_r[...] = acc[...] + _mm(u, w2k[...])


def _tc_tag1(rh, diss, acc, w2k):
  return pl.pallas_call(
      _tc_tag1_body,
      grid=(NB,),
      in_specs=[_half32, _row8, _row64, _full((H, H))],
      out_specs=_row64,
      out_shape=jax.ShapeDtypeStruct((N, H), jnp.float32),
  )(rh, diss, acc, w2k)


def _tc_final_body(rh, diss, acc, w23, b2, linw, linb, out_r):
  u = diss[:, :1] * _cat2(rh[...])
  s = jnp.maximum(acc[...] + _mm(u, w23[...]) + b2[...], 0.0)
  out_r[...] = _mm(s, linw[...]) + linb[...]


def _tc_final(rh, diss, acc, w23, b2, linw, linb):
  return pl.pallas_call(
      _tc_final_body,
      grid=(NB,),
      in_specs=[_half32, _row8, _row64, _full((H, H)), _full((1, H)),
                _full((H, 8)), _full((1, 8))],
      out_specs=_row8,
      out_shape=jax.ShapeDtypeStruct((N, 8), jnp.float32),
  )(rh, diss, acc, w23, b2, linw, linb)


def kernel(game_x, state_x, edge_index_v_v, edge_index_history_v_s,
           edge_attr_history_v_s, edge_index_in_v_s, edge_index_s_s,
           conv1_W, conv1_b, conv2_W, conv2_b,
           conv3_Wrel, conv3_brel, conv3_Wroot,
           conv4_Wl, conv4_bl, conv4_Wr, lin_W, lin_b):
  f32 = jnp.float32
  i32 = jnp.int32
  gx8 = jnp.pad(game_x.astype(f32), ((0, 0), (0, 3)))
  sx8 = jnp.pad(state_x.astype(f32), ((0, 0), (0, 2)))

  padr = jnp.zeros((EP,), i32)
  padc = jnp.full((EP,), NPAD - 1, i32)

  def prow(r):
    return jnp.concatenate([r.astype(i32), padr])

  def pcol(cl):
    return jnp.concatenate([cl.astype(i32), padc])

  rvv = prow(edge_index_v_v[0])
  cvv = pcol(edge_index_v_v[1])
  rh_ = prow(edge_index_history_v_s[0])
  ch = pcol(edge_index_history_v_s[1])
  rin = prow(edge_index_in_v_s[0])
  cin = pcol(edge_index_in_v_s[1])
  rss = prow(edge_index_s_s[0])
  css = pcol(edge_index_s_s[1])

  rvvB = jnp.concatenate([rvv, rvv])
  rhA = jnp.concatenate([rh_, rh_ + N])
  rinA = jnp.concatenate([rin, rin + N])
  rssA = jnp.concatenate([rss, rss + N])
  zrow2 = jnp.zeros((2 * EPAD,), i32)

  w1p = jnp.pad(conv1_W.astype(f32), ((0, 0), (0, 3), (0, 0)))
  w1h = [w1p[k].reshape(8, 2, 32).transpose(1, 0, 2) for k in range(3)]
  b1h = conv1_b.astype(f32).reshape(2, 1, 32)
  wrootp = jnp.pad(conv3_Wroot.astype(f32), ((0, 2), (0, 0)))
  w2 = conv2_W.astype(f32)

  z32 = jnp.zeros((RPT, 32), f32)
  ones2 = jnp.ones((2 * N, 32), f32)
  wz = jnp.zeros((EPAD,), f32)
  wh_ = jnp.concatenate([edge_attr_history_v_s.astype(f32),
                         jnp.zeros((EP,), f32)])

  def mkflag(wf, ma, hf):
    return jnp.array([wf, ma, hf] + [0] * 13, i32)

  f_hist = mkflag(0, 0, 1)
  f_b = mkflag(0, 0, 0)
  f_a = mkflag(0, 1, 0)
  f_aw = mkflag(1, 1, 0)

  hv = _sc_seg(ones2, zrow2, cvv, wz, f_hist, z32)
  hi = _sc_seg(ones2, zrow2, cin, wz, f_hist, z32)
  hs = _sc_seg(ones2, zrow2, css, wz, f_hist, z32)
  disg, xs1cat = _tc_prep_g(hv, gx8)
  diss, invc = _tc_prep_si(hi, hs)

  q1 = _sc_seg(xs1cat, rvvB, cvv, wz, f_b, z32)
  p1, xs2cat = _tc_rescale8(q1, disg)
  q2 = _sc_seg(xs2cat, rvvB, cvv, wz, f_b, z32)
  gq = _tc_conv1(q2, disg, gx8, p1, w1h[0], w1h[1], w1h[2], b1h)
  gqcat = gq.reshape(2 * N, 32)

  ag = _sc_seg(gqcat, rhA, ch, wh_, f_aw, z32)
  ss = _sc_seg(gqcat, rinA, cin, wz, f_a, z32)
  s4h = _tc_conv34(ag, ss, invc, sx8,
                   conv3_Wrel.astype(f32),
                   conv3_brel.astype(f32).reshape(1, H), wrootp,
                   conv4_Wl.astype(f32), conv4_bl.astype(f32).reshape(1, H),
                   conv4_Wr.astype(f32))

  t1cat = _tc_t_dense(s4h, diss).reshape(2 * N, 32)
  r1 = _sc_seg(t1cat, rssA, css, wz, f_a, z32)
  acc = _tc_tag0(r1, diss, s4h, w2[0], w2[1])
  t2cat = _tc_t_half(r1, diss).reshape(2 * N, 32)
  r2 = _sc_seg(t2cat, rssA, css, wz, f_a, z32)
  acc = _tc_tag1(r2, diss, acc, w2[2])
  t3cat = _tc_t_half(r2, diss).reshape(2 * N, 32)
  r3 = _sc_seg(t3cat, rssA, css, wz, f_a, z32)
  return _tc_final(r3, diss, acc, w2[3],
                   conv2_b.astype(f32).reshape(1, H),
                   lin_W.astype(f32), lin_b.astype(f32).reshape(1, 8))
